# baseline (speedup 1.0000x reference)
import functools

import jax
import jax.numpy as jnp
import numpy as np
from jax.experimental import pallas as pl
from jax.experimental.pallas import tpu as pltpu

_FREQ_WIDTH = 27
_TIME_WIDTH = 0.05

_F_STARTS = [[94, 9], [89, 5], [46, 3], [24, 63], [98, 71], [88, 51], [42, 51], [7, 82], [29, 31], [65, 31], [89, 29], [28, 85], [57, 61], [55, 62], [66, 75], [72, 43], [12, 53], [43, 58], [88, 39], [57, 19], [92, 50], [20, 90], [80, 47], [0, 20], [61, 84], [53, 61], [87, 69], [101, 90], [39, 31], [58, 9], [6, 31], [12, 36], [96, 12], [75, 21], [23, 14], [52, 18], [35, 62], [10, 63], [52, 24], [19, 72], [94, 40], [76, 17], [85, 53], [82, 3], [81, 13], [8, 63], [59, 74], [23, 25], [96, 27], [17, 3], [55, 23], [85, 82], [83, 16], [45, 63], [4, 33], [66, 35], [62, 90], [19, 32], [26, 49], [14, 80], [19, 66], [76, 68], [101, 65], [31, 69]]

_T_STARTS = [[1934, 446, 1804, 584, 1654], [1242, 982, 1093, 1865, 487], [1151, 1260, 789, 1656, 1254], [18, 501, 1636, 187, 1345], [827, 1275, 1795, 185, 690], [920, 196, 932, 1937, 1353], [864, 694, 1914, 846, 1885], [1627, 1306, 1698, 395, 605], [106, 679, 1671, 460, 334], [409, 1443, 1452, 1865, 482], [956, 1034, 309, 1497, 1375], [167, 600, 930, 34, 680], [1665, 1595, 1521, 459, 378], [191, 1943, 355, 480, 919], [39, 1229, 218, 1723, 1902], [1655, 108, 717, 120, 627], [1004, 462, 1569, 1301, 1374], [1178, 1592, 1072, 456, 104], [779, 889, 1258, 287, 299], [328, 400, 1614, 1758, 1085], [1789, 340, 1427, 1248, 1428], [176, 185, 21, 1497, 1357], [228, 1019, 675, 1196, 865], [310, 908, 1161, 800, 30], [583, 1608, 1574, 291, 275], [1541, 1631, 1804, 174, 850], [488, 659, 1860, 470, 977], [1063, 1200, 50, 342, 1116], [716, 1417, 1229, 1877, 268], [1632, 1905, 1849, 975, 447], [523, 723, 1610, 566, 909], [695, 20, 657, 497, 1211], [1022, 223, 73, 83, 978], [1627, 1498, 241, 1403, 768], [1336, 1740, 1010, 527, 1270], [1077, 1898, 143, 1503, 1933], [185, 774, 29, 57, 1483], [935, 1469, 1757, 474, 17], [981, 806, 524, 170, 307], [1080, 125, 1747, 106, 746], [1729, 252, 555, 644, 810], [761, 1286, 1564, 1031, 1126], [464, 895, 1847, 1732, 1765], [259, 464, 466, 1038, 1177], [1871, 905, 202, 90, 307], [745, 151, 871, 1084, 554], [191, 1079, 1921, 103, 1577], [873, 1729, 624, 1873, 1764], [68, 1628, 867, 447, 737], [1810, 627, 1892, 641, 236], [1379, 1305, 481, 0, 1765], [1498, 1494, 289, 629, 1769], [1486, 488, 1101, 1637, 3], [1486, 691, 975, 1094, 253], [671, 1584, 1859, 1462, 303], [944, 704, 429, 1118, 1225], [1271, 1303, 1248, 1136, 18], [1558, 786, 1536, 1737, 1357], [247, 610, 156, 1025, 1116], [311, 1695, 1041, 1559, 1651], [1702, 871, 297, 534, 954], [1487, 1346, 1136, 334, 1804], [1096, 1663, 853, 196, 224], [1643, 903, 1234, 1795, 386]]

_NSLOTS = 3


def _runs(mask):
    out = []
    i, n = 0, len(mask)
    while i < n:
        if mask[i]:
            j = i
            while j < n and mask[j]:
                j += 1
            out.append((i, j))
            i = j
        else:
            i += 1
    return out


def _row_geometry(B, F):
    masked, unmasked = [], []
    for b in range(B):
        fm = np.zeros(F, bool)
        for s in _F_STARTS[b]:
            fm[s:s + _FREQ_WIDTH] = True
        masked.append(_runs(fm))
        unmasked.append(_runs(~fm))
    return masked, unmasked


def _body(x_ref, o_ref, buf, zeros, sem_in, sem_out, sem_fill,
          *, B, F, T, tw, masked, unmasked):
    lanes = 128
    sub = T // lanes
    zeros[...] = jnp.zeros_like(zeros)

    gathers = {}
    scatters = {}
    fill_descs = []

    def issue_gathers(b):
        slot = b % _NSLOTS
        lst = []
        off = 0
        for r0, r1 in unmasked[b]:
            n = r1 - r0
            cp = pltpu.make_async_copy(
                x_ref.at[pl.ds(b * F + r0, n)],
                buf.at[slot, pl.ds(off, n)],
                sem_in.at[slot],
            )
            cp.start()
            lst.append(cp)
            off += n
        gathers[b] = lst

    def issue_fills(b):
        for r0, r1 in masked[b]:
            n = r1 - r0
            cp = pltpu.make_async_copy(
                zeros.at[pl.ds(0, n)],
                o_ref.at[pl.ds(b * F + r0, n)],
                sem_fill,
            )
            cp.start()
            fill_descs.append(cp)

    def issue_scatters(b):
        slot = b % _NSLOTS
        lst = []
        off = 0
        for r0, r1 in unmasked[b]:
            n = r1 - r0
            cp = pltpu.make_async_copy(
                buf.at[slot, pl.ds(off, n)],
                o_ref.at[pl.ds(b * F + r0, n)],
                sem_out.at[slot],
            )
            cp.start()
            lst.append(cp)
            off += n
        scatters[b] = lst

    def compute(b):
        slot = b % _NSLOTS
        nrows = sum(r1 - r0 for r0, r1 in unmasked[b])
        ci = (jax.lax.broadcasted_iota(jnp.int32, (1, sub, lanes), 1) * lanes
              + jax.lax.broadcasted_iota(jnp.int32, (1, sub, lanes), 2))
        tm = functools.reduce(
            jnp.logical_or,
            [(ci >= s) & (ci < s + tw) for s in _T_STARTS[b]],
        )
        v = buf[slot, 0:nrows]
        buf[slot, 0:nrows] = jnp.where(tm, jnp.float32(0.0), v)

    issue_gathers(0)
    for b in range(B):
        if b + 1 < B:
            if b - 2 >= 0:
                for cp in scatters[b - 2]:
                    cp.wait()
            issue_gathers(b + 1)
        issue_fills(b)
        for cp in gathers[b]:
            cp.wait()
        compute(b)
        issue_scatters(b)
    for b in (B - 3, B - 2, B - 1):
        for cp in scatters[b]:
            cp.wait()
    for cp in fill_descs:
        cp.wait()


def kernel(input_spec):
    B, F, T = input_spec.shape
    tw = int(_TIME_WIDTH * T)
    lanes = 128
    sub = T // lanes
    masked, unmasked = _row_geometry(B, F)
    max_fill = max(r1 - r0 for per_b in masked for r0, r1 in per_b)
    max_rows = max(sum(r1 - r0 for r0, r1 in per_b) for per_b in unmasked)

    x3 = input_spec.reshape(B * F, sub, lanes)
    body = functools.partial(
        _body, B=B, F=F, T=T, tw=tw, masked=masked, unmasked=unmasked)
    out = pl.pallas_call(
        body,
        in_specs=[pl.BlockSpec(memory_space=pl.ANY)],
        out_specs=pl.BlockSpec(memory_space=pl.ANY),
        out_shape=jax.ShapeDtypeStruct((B * F, sub, lanes), input_spec.dtype),
        scratch_shapes=[
            pltpu.VMEM((_NSLOTS, max_rows, sub, lanes), jnp.float32),
            pltpu.VMEM((max_fill, sub, lanes), jnp.float32),
            pltpu.SemaphoreType.DMA((_NSLOTS,)),
            pltpu.SemaphoreType.DMA((_NSLOTS,)),
            pltpu.SemaphoreType.DMA,
        ],
    )(x3)
    return out.reshape(B, F, T)


# speedup vs baseline: 1.1226x; 1.1226x over previous
import functools

import jax
import jax.numpy as jnp
import numpy as np
from jax.experimental import pallas as pl
from jax.experimental.pallas import tpu as pltpu

_FREQ_WIDTH = 27
_TIME_WIDTH = 0.05

_F_STARTS = [[94, 9], [89, 5], [46, 3], [24, 63], [98, 71], [88, 51], [42, 51], [7, 82], [29, 31], [65, 31], [89, 29], [28, 85], [57, 61], [55, 62], [66, 75], [72, 43], [12, 53], [43, 58], [88, 39], [57, 19], [92, 50], [20, 90], [80, 47], [0, 20], [61, 84], [53, 61], [87, 69], [101, 90], [39, 31], [58, 9], [6, 31], [12, 36], [96, 12], [75, 21], [23, 14], [52, 18], [35, 62], [10, 63], [52, 24], [19, 72], [94, 40], [76, 17], [85, 53], [82, 3], [81, 13], [8, 63], [59, 74], [23, 25], [96, 27], [17, 3], [55, 23], [85, 82], [83, 16], [45, 63], [4, 33], [66, 35], [62, 90], [19, 32], [26, 49], [14, 80], [19, 66], [76, 68], [101, 65], [31, 69]]

_T_STARTS = [[1934, 446, 1804, 584, 1654], [1242, 982, 1093, 1865, 487], [1151, 1260, 789, 1656, 1254], [18, 501, 1636, 187, 1345], [827, 1275, 1795, 185, 690], [920, 196, 932, 1937, 1353], [864, 694, 1914, 846, 1885], [1627, 1306, 1698, 395, 605], [106, 679, 1671, 460, 334], [409, 1443, 1452, 1865, 482], [956, 1034, 309, 1497, 1375], [167, 600, 930, 34, 680], [1665, 1595, 1521, 459, 378], [191, 1943, 355, 480, 919], [39, 1229, 218, 1723, 1902], [1655, 108, 717, 120, 627], [1004, 462, 1569, 1301, 1374], [1178, 1592, 1072, 456, 104], [779, 889, 1258, 287, 299], [328, 400, 1614, 1758, 1085], [1789, 340, 1427, 1248, 1428], [176, 185, 21, 1497, 1357], [228, 1019, 675, 1196, 865], [310, 908, 1161, 800, 30], [583, 1608, 1574, 291, 275], [1541, 1631, 1804, 174, 850], [488, 659, 1860, 470, 977], [1063, 1200, 50, 342, 1116], [716, 1417, 1229, 1877, 268], [1632, 1905, 1849, 975, 447], [523, 723, 1610, 566, 909], [695, 20, 657, 497, 1211], [1022, 223, 73, 83, 978], [1627, 1498, 241, 1403, 768], [1336, 1740, 1010, 527, 1270], [1077, 1898, 143, 1503, 1933], [185, 774, 29, 57, 1483], [935, 1469, 1757, 474, 17], [981, 806, 524, 170, 307], [1080, 125, 1747, 106, 746], [1729, 252, 555, 644, 810], [761, 1286, 1564, 1031, 1126], [464, 895, 1847, 1732, 1765], [259, 464, 466, 1038, 1177], [1871, 905, 202, 90, 307], [745, 151, 871, 1084, 554], [191, 1079, 1921, 103, 1577], [873, 1729, 624, 1873, 1764], [68, 1628, 867, 447, 737], [1810, 627, 1892, 641, 236], [1379, 1305, 481, 0, 1765], [1498, 1494, 289, 629, 1769], [1486, 488, 1101, 1637, 3], [1486, 691, 975, 1094, 253], [671, 1584, 1859, 1462, 303], [944, 704, 429, 1118, 1225], [1271, 1303, 1248, 1136, 18], [1558, 786, 1536, 1737, 1357], [247, 610, 156, 1025, 1116], [311, 1695, 1041, 1559, 1651], [1702, 871, 297, 534, 954], [1487, 1346, 1136, 334, 1804], [1096, 1663, 853, 196, 224], [1643, 903, 1234, 1795, 386]]

def _runs(mask):
    out = []
    i, n = 0, len(mask)
    while i < n:
        if mask[i]:
            j = i
            while j < n and mask[j]:
                j += 1
            out.append((i, j))
            i = j
        else:
            i += 1
    return out


def _row_geometry(B, F):
    masked, unmasked = [], []
    for b in range(B):
        fm = np.zeros(F, bool)
        for s in _F_STARTS[b]:
            fm[s:s + _FREQ_WIDTH] = True
        masked.append(_runs(fm))
        unmasked.append(_runs(~fm))
    return masked, unmasked


def _body(x_ref, o_ref, buf, zeros, sem_in, sem_out, sem_fill,
          *, B, F, T, tw, masked, unmasked, offsets):
    lanes = 128
    sub = T // lanes
    zeros[...] = jnp.zeros_like(zeros)

    gathers = {}
    scatters = []
    fills = []

    for b in range(B):
        lst = []
        off = offsets[b]
        for r0, r1 in unmasked[b]:
            n = r1 - r0
            cp = pltpu.make_async_copy(
                x_ref.at[pl.ds(b * F + r0, n)],
                buf.at[pl.ds(off, n)],
                sem_in.at[b],
            )
            cp.start()
            lst.append(cp)
            off += n
        gathers[b] = lst
    for b in range(B):
        for r0, r1 in masked[b]:
            n = r1 - r0
            cp = pltpu.make_async_copy(
                zeros.at[pl.ds(0, n)],
                o_ref.at[pl.ds(b * F + r0, n)],
                sem_fill,
            )
            cp.start()
            fills.append(cp)

    ci = (jax.lax.broadcasted_iota(jnp.int32, (1, sub, lanes), 1) * lanes
          + jax.lax.broadcasted_iota(jnp.int32, (1, sub, lanes), 2))
    for b in range(B):
        for cp in gathers[b]:
            cp.wait()
        nrows = offsets[b + 1] - offsets[b]
        tm = functools.reduce(
            jnp.logical_or,
            [(ci >= s) & (ci < s + tw) for s in _T_STARTS[b]],
        )
        region = slice(offsets[b], offsets[b] + nrows)
        buf[region] = jnp.where(tm, jnp.float32(0.0), buf[region])
        off = offsets[b]
        for r0, r1 in unmasked[b]:
            n = r1 - r0
            cp = pltpu.make_async_copy(
                buf.at[pl.ds(off, n)],
                o_ref.at[pl.ds(b * F + r0, n)],
                sem_out,
            )
            cp.start()
            scatters.append(cp)
            off += n
    for cp in scatters:
        cp.wait()
    for cp in fills:
        cp.wait()


def kernel(input_spec):
    B, F, T = input_spec.shape
    tw = int(_TIME_WIDTH * T)
    lanes = 128
    sub = T // lanes
    masked, unmasked = _row_geometry(B, F)
    max_fill = max(r1 - r0 for per_b in masked for r0, r1 in per_b)
    offsets = [0]
    for per_b in unmasked:
        offsets.append(offsets[-1] + sum(r1 - r0 for r0, r1 in per_b))
    total_rows = offsets[-1]

    x3 = input_spec.reshape(B * F, sub, lanes)
    body = functools.partial(
        _body, B=B, F=F, T=T, tw=tw, masked=masked, unmasked=unmasked,
        offsets=offsets)
    out = pl.pallas_call(
        body,
        in_specs=[pl.BlockSpec(memory_space=pl.ANY)],
        out_specs=pl.BlockSpec(memory_space=pl.ANY),
        out_shape=jax.ShapeDtypeStruct((B * F, sub, lanes), input_spec.dtype),
        scratch_shapes=[
            pltpu.VMEM((total_rows, sub, lanes), jnp.float32),
            pltpu.VMEM((max_fill, sub, lanes), jnp.float32),
            pltpu.SemaphoreType.DMA((B,)),
            pltpu.SemaphoreType.DMA,
            pltpu.SemaphoreType.DMA,
        ],
    )(x3)
    return out.reshape(B, F, T)


# speedup vs baseline: 2.5335x; 2.2568x over previous
import functools

import jax
import jax.numpy as jnp
from jax.experimental import pallas as pl
from jax.experimental.pallas import tpu as pltpu

_FREQ_MASKS = 2
_TIME_MASKS = 5
_FREQ_WIDTH = 27
_TIME_WIDTH = 0.05

_SAMPLES_PER_BLOCK = 4


def _mask_starts(B, F, T, tw):
    k = jax.random.key(42)
    kf, kt = jax.random.split(k)
    f_starts = jax.random.randint(kf, (B, _FREQ_MASKS), 0, F - _FREQ_WIDTH + 1)
    t_starts = jax.random.randint(kt, (B, _TIME_MASKS), 0, T - tw + 1)
    return f_starts.astype(jnp.int32), t_starts.astype(jnp.int32)


def _body(fs_ref, ts_ref, x_ref, o_ref, *, F, T, tw):
    j = pl.program_id(0)
    fi = jax.lax.broadcasted_iota(jnp.int32, (F, 1), 0)
    ti = jax.lax.broadcasted_iota(jnp.int32, (1, T), 1)
    for s in range(_SAMPLES_PER_BLOCK):
        b = j * _SAMPLES_PER_BLOCK + s
        fm = jnp.zeros((F, 1), jnp.bool_)
        for m in range(_FREQ_MASKS):
            st = fs_ref[b, m]
            fm = fm | ((fi >= st) & (fi < st + _FREQ_WIDTH))
        tm = jnp.zeros((1, T), jnp.bool_)
        for m in range(_TIME_MASKS):
            st = ts_ref[b, m]
            tm = tm | ((ti >= st) & (ti < st + tw))
        rows = slice(s * F, (s + 1) * F)
        o_ref[rows, :] = jnp.where(fm | tm, jnp.float32(0.0), x_ref[rows, :])


def kernel(input_spec):
    B, F, T = input_spec.shape
    tw = int(_TIME_WIDTH * T)
    f_starts, t_starts = _mask_starts(B, F, T, tw)
    x2 = input_spec.reshape(B * F, T)
    block_rows = _SAMPLES_PER_BLOCK * F
    grid = (B // _SAMPLES_PER_BLOCK,)

    body = functools.partial(_body, F=F, T=T, tw=tw)
    out = pl.pallas_call(
        body,
        grid_spec=pltpu.PrefetchScalarGridSpec(
            num_scalar_prefetch=2,
            grid=grid,
            in_specs=[pl.BlockSpec((block_rows, T), lambda j, fs, ts: (j, 0))],
            out_specs=pl.BlockSpec((block_rows, T), lambda j, fs, ts: (j, 0)),
        ),
        out_shape=jax.ShapeDtypeStruct((B * F, T), input_spec.dtype),
        compiler_params=pltpu.CompilerParams(
            dimension_semantics=("arbitrary",),
        ),
    )(f_starts, t_starts, x2)
    return out.reshape(B, F, T)
